# baseline (speedup 1.0000x reference)
import functools

import jax
import jax.numpy as jnp
from jax import lax
from jax.experimental import pallas as pl
from jax.experimental.pallas import tpu as pltpu
from jax.experimental.pallas import tpu_sc as plsc

NC = 2
NS = 16
NW = NC * NS
CHUNK = 128
NBUF = 8


def _make_lookup(B, H, D):
    bpw = B // NW
    kph = bpw // CHUNK
    n_iters = H * kph
    assert B % (NW * CHUNK) == 0 and n_iters % NBUF == 0 and D == 64
    n_groups = n_iters // NBUF
    DR = D // 8
    CT = B // CHUNK
    mesh = plsc.VectorSubcoreMesh(
        core_axis_name="c", subcore_axis_name="s",
        num_cores=NC, num_subcores=NS,
    )

    @functools.partial(
        pl.kernel,
        mesh=mesh,
        out_type=jax.ShapeDtypeStruct((H, DR, CT, 8, CHUNK), jnp.float32),
        scratch_types=[
            pltpu.VMEM((H, bpw), jnp.int32),
            pltpu.VMEM((NBUF, CHUNK, D), jnp.float32),
            pltpu.VMEM((2, D, CHUNK), jnp.float32),
            pltpu.SemaphoreType.DMA((NBUF,)),
            pltpu.SemaphoreType.DMA((2,)),
        ],
        compiler_params=pltpu.CompilerParams(
            use_tc_tiling_on_sc=False, needs_layout_passes=False
        ),
    )
    def run(idx_hbm, table_hbm, out_hbm, idx_v, rows_v, xbuf, gsem, ssem):
        wid = lax.axis_index("s") * NC + lax.axis_index("c")
        b0 = wid * bpw
        c0 = wid * kph
        pltpu.sync_copy(idx_hbm.at[:, pl.ds(b0, bpw)], idx_v)
        lane = lax.iota(jnp.int32, 16)

        def gather_start(j, b):
            h = j // kph
            k = j % kph
            pltpu.async_copy(
                table_hbm.at[idx_v.at[h, pl.ds(k * CHUNK, CHUNK)]],
                rows_v.at[b],
                gsem.at[b],
            )

        def gather_wait(b):
            pltpu.make_async_copy(
                table_hbm.at[pl.ds(0, CHUNK)], rows_v.at[b], gsem.at[b]
            ).wait()

        def transpose(b, q):
            src = rows_v.at[b]
            dst = xbuf.at[q]

            def xp(d, _):
                col = jnp.full((16,), d, jnp.int32)
                for lb in range(CHUNK // 16):
                    v = plsc.load_gather(src, [lane + lb * 16, col])
                    dst[d, pl.ds(lb * 16, 16)] = v
                return 0

            lax.fori_loop(0, D, xp, 0)

        def stores_start(j, q):
            h = j // kph
            k = j % kph
            for r in range(DR):
                pltpu.async_copy(
                    xbuf.at[q, pl.ds(r * 8, 8)],
                    out_hbm.at[h, r, c0 + k],
                    ssem.at[q],
                )

        def stores_wait(q):
            pltpu.make_async_copy(
                table_hbm.at[pl.ds(0, CHUNK)], rows_v.at[0], ssem.at[q]
            ).wait()

        for b in range(NBUF):
            gather_start(b, b)

        for b in range(NBUF):
            q = b % 2
            gather_wait(b)
            if b >= 2:
                stores_wait(q)
            transpose(b, q)
            gather_start(b + NBUF, b)
            stores_start(b, q)

        def group(g, _):
            for b in range(NBUF):
                j = g * NBUF + b
                q = b % 2
                gather_wait(b)
                stores_wait(q)
                transpose(b, q)
                gather_start(j + NBUF, b)
                stores_start(j, q)
            return 0

        lax.fori_loop(1, n_groups - 1, group, 0)

        for b in range(NBUF):
            j = (n_groups - 1) * NBUF + b
            q = b % 2
            gather_wait(b)
            stores_wait(q)
            transpose(b, q)
            stores_start(j, q)
        for q in range(2):
            stores_wait(q)

    return run


def kernel(input_tokens, table):
    B, H = input_tokens.shape
    V, D = table.shape
    idx_t = input_tokens.T.astype(jnp.int32)
    x = _make_lookup(B, H, D)(idx_t, table)
    return x.transpose((2, 4, 0, 1, 3)).reshape(B, H, D)


# speedup vs baseline: 1.4573x; 1.4573x over previous
import functools

import jax
import jax.numpy as jnp
from jax import lax
from jax.experimental import pallas as pl
from jax.experimental.pallas import tpu as pltpu
from jax.experimental.pallas import tpu_sc as plsc

NC = 2
NS = 16
NW = NC * NS
CHUNK = 128
NBUF = 8
LAG = 4


def _make_lookup(B, H, D):
    total = B * H
    K = total // CHUNK
    kpt = K // NW
    n_groups = kpt // NBUF
    assert total % (NW * CHUNK) == 0 and kpt % NBUF == 0 and n_groups >= 3
    mesh = plsc.VectorSubcoreMesh(
        core_axis_name="c", subcore_axis_name="s",
        num_cores=NC, num_subcores=NS,
    )

    @functools.partial(
        pl.kernel,
        mesh=mesh,
        out_type=jax.ShapeDtypeStruct((K, CHUNK, D), jnp.float32),
        scratch_types=[
            pltpu.VMEM((kpt, CHUNK), jnp.int32),
            pltpu.VMEM((NBUF, CHUNK, D), jnp.float32),
            pltpu.SemaphoreType.DMA((NBUF,)),
            pltpu.SemaphoreType.DMA((NBUF,)),
        ],
        compiler_params=pltpu.CompilerParams(
            use_tc_tiling_on_sc=False, needs_layout_passes=False
        ),
    )
    def run(idx_hbm, table_hbm, out_hbm, idx_v, rows_v, gsem, ssem):
        wid = lax.axis_index("s") * NC + lax.axis_index("c")
        k0 = wid * kpt
        pltpu.sync_copy(idx_hbm.at[pl.ds(k0, kpt)], idx_v)

        def gather_start(k, b):
            pltpu.async_copy(
                table_hbm.at[idx_v.at[k]], rows_v.at[b], gsem.at[b]
            )

        def gather_wait(b):
            pltpu.make_async_copy(
                table_hbm.at[pl.ds(0, CHUNK)], rows_v.at[b], gsem.at[b]
            ).wait()

        def store_start(k, b):
            pltpu.async_copy(rows_v.at[b], out_hbm.at[k0 + k], ssem.at[b])

        def store_wait(b):
            pltpu.make_async_copy(
                table_hbm.at[pl.ds(0, CHUNK)], rows_v.at[b], ssem.at[b]
            ).wait()

        for b in range(NBUF):
            gather_start(b, b)

        for k in range(NBUF):
            gather_wait(k)
            store_start(k, k)
            if k >= LAG:
                b2 = k - LAG
                store_wait(b2)
                gather_start(k - LAG + NBUF, b2)

        def group(g, _):
            for b in range(NBUF):
                k = g * NBUF + b
                gather_wait(b)
                store_start(k, b)
                b2 = (b - LAG) % NBUF
                store_wait(b2)
                gather_start(k - LAG + NBUF, b2)
            return 0

        lax.fori_loop(1, n_groups - 1, group, 0)

        for b in range(NBUF):
            k = (n_groups - 1) * NBUF + b
            gather_wait(b)
            store_start(k, b)
            if b < LAG:
                b2 = (b - LAG) % NBUF
                store_wait(b2)
                gather_start(k - LAG + NBUF, b2)
        for b in range(NBUF):
            store_wait(b)

    return run


def kernel(input_tokens, table):
    B, H = input_tokens.shape
    V, D = table.shape
    idx = input_tokens.astype(jnp.int32).reshape(-1, CHUNK)
    x = _make_lookup(B, H, D)(idx, table)
    return x.reshape(B, H, D)
